# Initial kernel scaffold; baseline (speedup 1.0000x reference)
import jax
import jax.numpy as jnp
from jax.experimental import pallas as pl


def kernel(x, pos):
    raise NotImplementedError("write your pallas kernel here")


# speedup vs baseline: 2.0559x; 2.0559x over previous
import functools

import jax
import jax.numpy as jnp
from jax.experimental import pallas as pl

N = 8192
DIM = 64
H = 4
E = 16
M = 64
NBALLS = N // M
SCALE = DIM ** (-0.5)
ASCALE = E ** (-0.5)
NEG = -1e30


def _prep_body(x_ref, pos_ref, xeh_ref, kmean_ref):
    xb = x_ref[...]
    pb = pos_ref[...]
    xe = xb + pb - jnp.mean(pb, axis=0, keepdims=True)
    for h in range(H):
        xeh_ref[h, :, :] = xe[:, h * E:(h + 1) * E]
    kmean_ref[0, :, :] = jnp.mean(xe, axis=0, keepdims=True)


def _attn_body(kmean_ref, xe_ref, q_ref, o_ref, *, bq, kc):
    q = q_ref[0]
    km = kmean_ref[0]

    logits = jax.lax.dot_general(
        q, km, (((1,), (1,)), ((), ())),
        preferred_element_type=jnp.float32) * SCALE
    cols = jax.lax.broadcasted_iota(jnp.int32, (bq, NBALLS), 1)
    m1 = jnp.max(logits, axis=1, keepdims=True)
    i1 = jnp.min(jnp.where(logits >= m1, cols, NBALLS), axis=1, keepdims=True)
    l2 = jnp.where(cols == i1, NEG, logits)
    m2 = jnp.max(l2, axis=1, keepdims=True)
    i2 = jnp.min(jnp.where(l2 >= m2, cols, NBALLS), axis=1, keepdims=True)

    nchunks = N // kc
    m_run = jnp.full((bq, 1), NEG, dtype=jnp.float32)
    l_run = jnp.zeros((bq, 1), dtype=jnp.float32)
    o_run = jnp.zeros((bq, E), dtype=jnp.float32)

    def step(c, carry):
        m_run, l_run, o_run = carry
        kv = xe_ref[0, pl.ds(c * kc, kc), :]
        s = jax.lax.dot_general(
            q, kv, (((1,), (1,)), ((), ())),
            preferred_element_type=jnp.float32) * ASCALE
        ball = (c * kc + jax.lax.broadcasted_iota(jnp.int32, (bq, kc), 1)) // M
        sel = (ball == i1) | (ball == i2)
        s = jnp.where(sel, s, NEG)
        m_new = jnp.maximum(m_run, jnp.max(s, axis=1, keepdims=True))
        p = jnp.where(sel, jnp.exp(s - m_new), 0.0)
        corr = jnp.exp(m_run - m_new)
        l_new = l_run * corr + jnp.sum(p, axis=1, keepdims=True)
        o_new = o_run * corr + jax.lax.dot_general(
            p, kv, (((1,), (0,)), ((), ())),
            preferred_element_type=jnp.float32)
        return m_new, l_new, o_new

    m_run, l_run, o_run = jax.lax.fori_loop(
        0, nchunks, step, (m_run, l_run, o_run))
    o_ref[0, :, :] = o_run / l_run


@jax.jit
def kernel(x, pos):
    xeh, kmean = pl.pallas_call(
        _prep_body,
        grid=(NBALLS,),
        in_specs=[
            pl.BlockSpec((M, DIM), lambda b: (b, 0)),
            pl.BlockSpec((M, DIM), lambda b: (b, 0)),
        ],
        out_specs=[
            pl.BlockSpec((H, M, E), lambda b: (0, b, 0)),
            pl.BlockSpec((1, 1, DIM), lambda b: (b, 0, 0)),
        ],
        out_shape=[
            jax.ShapeDtypeStruct((H, N, E), jnp.float32),
            jax.ShapeDtypeStruct((NBALLS, 1, DIM), jnp.float32),
        ],
    )(x, pos)
    kmeanh = jnp.transpose(
        kmean.reshape(NBALLS, H, E), (1, 0, 2))

    bq = 512
    kc = 1024
    out = pl.pallas_call(
        functools.partial(_attn_body, bq=bq, kc=kc),
        grid=(N // bq, H),
        in_specs=[
            pl.BlockSpec((1, NBALLS, E), lambda qi, h: (h, 0, 0)),
            pl.BlockSpec((1, N, E), lambda qi, h: (h, 0, 0)),
            pl.BlockSpec((1, bq, E), lambda qi, h: (h, qi, 0)),
        ],
        out_specs=pl.BlockSpec((1, bq, E), lambda qi, h: (h, qi, 0)),
        out_shape=jax.ShapeDtypeStruct((H, N, E), jnp.float32),
    )(kmeanh, xeh, xeh)
    return jnp.transpose(out, (1, 0, 2))
